# baseline (device time: 109468 ns/iter reference)
import jax
import jax.numpy as jnp
from jax import lax
from jax.experimental import pallas as pl
from jax.experimental.pallas import tpu as pltpu

N_RING = 8
MC = 512
PS = 384
SUB = 192
PX = 128
NH = 4


def kernel(x):
    m, n = x.shape
    assert m == N_RING * MC

    def body(x_ref, out_ref, p1_buf,
             p1_send, p1_recv, fwd_send, fwd_recv,
             bwd_send, bwd_recv, xf_send, xf_recv):
        my_x = lax.axis_index("x")
        my_y = lax.axis_index("y")
        my_z = lax.axis_index("z")
        partner = (1 - my_x, my_y, my_z)

        r = jnp.where(my_y == 0, my_z, (N_RING - 1) - my_z)

        def pos_coords(p):
            p = p % N_RING
            ny = jnp.where(p < 4, 0, 1)
            nz = jnp.where(p < 4, p, (N_RING - 1) - p)
            return ny, nz

        fy, fz = pos_coords(r + 1)
        by, bz = pos_coords(r - 1)
        fwd = (my_x, fy, fz)
        bwd = (my_x, by, bz)

        off_s = my_x * 128
        off_sp = (1 - my_x) * 128
        off_xs = my_x * 384

        barrier_sem = pltpu.get_barrier_semaphore()
        for nbr in (partner, fwd, bwd):
            pl.semaphore_signal(
                barrier_sem, inc=1,
                device_id=nbr, device_id_type=pl.DeviceIdType.MESH,
            )
        pl.semaphore_wait(barrier_sem, 3)

        row0 = r * MC

        p1_rdmas = []
        for s in range(2):
            p1_rdmas.append(pltpu.make_async_remote_copy(
                src_ref=x_ref.at[pl.ds(row0 + off_sp + s * SUB, SUB)],
                dst_ref=p1_buf.at[pl.ds(s * SUB, SUB)],
                send_sem=p1_send.at[s], recv_sem=p1_recv.at[s],
                device_id=partner, device_id_type=pl.DeviceIdType.MESH,
            ))
            p1_rdmas[s].start()

        def mk_hop(h, s, sign, sems_s, sems_r, dev):
            c = (r + sign * (h - 1)) % N_RING
            rows = pl.ds(c * MC + off_s + s * SUB, SUB)
            i = 2 * (h - 1) + s
            return pltpu.make_async_remote_copy(
                src_ref=out_ref.at[rows], dst_ref=out_ref.at[rows],
                send_sem=sems_s.at[i], recv_sem=sems_r.at[i],
                device_id=dev, device_id_type=pl.DeviceIdType.MESH,
            )

        def mk_xfwd(slot, c):
            rows = pl.ds(c * MC + off_xs, PX)
            return pltpu.make_async_remote_copy(
                src_ref=out_ref.at[rows], dst_ref=out_ref.at[rows],
                send_sem=xf_send.at[slot], recv_sem=xf_recv.at[slot],
                device_id=partner, device_id_type=pl.DeviceIdType.MESH,
            )

        fwd_rdmas = {}
        bwd_rdmas = {}
        x_rdmas = []

        for s in range(2):
            p1_rdmas[s].wait_recv()
            out_ref[pl.ds(row0 + off_s + s * SUB, SUB), :] = (
                x_ref[pl.ds(row0 + off_s + s * SUB, SUB), :]
                + p1_buf[pl.ds(s * SUB, SUB), :]
            )
            fwd_rdmas[(1, s)] = mk_hop(1, s, -1, fwd_send, fwd_recv, fwd)
            fwd_rdmas[(1, s)].start()
            bwd_rdmas[(1, s)] = mk_hop(1, s, +1, bwd_send, bwd_recv, bwd)
            bwd_rdmas[(1, s)].start()
        x_rdmas.append(mk_xfwd(0, r))
        x_rdmas[-1].start()

        for rd in p1_rdmas:
            rd.wait_send()

        for h in range(1, NH + 1):
            subs_f = (0, 1) if h < NH else (0,)
            subs_b = (0, 1) if h < NH else (1,)
            for s in subs_f:
                fwd_rdmas[(h, s)].wait_recv()
                if h < NH and (h + 1 < NH or s == 0):
                    fwd_rdmas[(h + 1, s)] = mk_hop(
                        h + 1, s, -1, fwd_send, fwd_recv, fwd)
                    fwd_rdmas[(h + 1, s)].start()
            if h < NH:
                x_rdmas.append(mk_xfwd(2 * h - 1, (r - h) % N_RING))
                x_rdmas[-1].start()
            for s in subs_b:
                bwd_rdmas[(h, s)].wait_recv()
                if h < NH and (h + 1 < NH or s == 1):
                    bwd_rdmas[(h + 1, s)] = mk_hop(
                        h + 1, s, +1, bwd_send, bwd_recv, bwd)
                    bwd_rdmas[(h + 1, s)].start()
            if h < NH:
                x_rdmas.append(mk_xfwd(2 * h, (r + h) % N_RING))
                x_rdmas[-1].start()
            for s in subs_f:
                fwd_rdmas[(h, s)].wait_send()
            for s in subs_b:
                bwd_rdmas[(h, s)].wait_send()

        x_rdmas.append(mk_xfwd(7, (r + NH) % N_RING))
        x_rdmas[-1].start()

        for rd in x_rdmas:
            rd.wait_recv()
            rd.wait_send()

    return pl.pallas_call(
        body,
        out_shape=jax.ShapeDtypeStruct((m, n), x.dtype),
        in_specs=[pl.BlockSpec(memory_space=pltpu.VMEM)],
        out_specs=pl.BlockSpec(memory_space=pltpu.VMEM),
        scratch_shapes=[
            pltpu.VMEM((PS, n), x.dtype),
            pltpu.SemaphoreType.DMA((2,)),
            pltpu.SemaphoreType.DMA((2,)),
            pltpu.SemaphoreType.DMA((2 * NH,)),
            pltpu.SemaphoreType.DMA((2 * NH,)),
            pltpu.SemaphoreType.DMA((2 * NH,)),
            pltpu.SemaphoreType.DMA((2 * NH,)),
            pltpu.SemaphoreType.DMA((N_RING,)),
            pltpu.SemaphoreType.DMA((N_RING,)),
        ],
        compiler_params=pltpu.CompilerParams(
            collective_id=0,
            vmem_limit_bytes=100 * 1024 * 1024,
        ),
    )(x)


# device time: 109257 ns/iter; 1.0019x vs baseline; 1.0019x over previous
import jax
import jax.numpy as jnp
from jax import lax
from jax.experimental import pallas as pl
from jax.experimental.pallas import tpu as pltpu

N_RING = 8
MC = 512
PS = 336
SUB = 168
PX = 176
NH = 4


def kernel(x):
    m, n = x.shape
    assert m == N_RING * MC

    def body(x_ref, out_ref, p1_buf,
             p1_send, p1_recv, fwd_send, fwd_recv,
             bwd_send, bwd_recv, xf_send, xf_recv):
        my_x = lax.axis_index("x")
        my_y = lax.axis_index("y")
        my_z = lax.axis_index("z")
        partner = (1 - my_x, my_y, my_z)

        r = jnp.where(my_y == 0, my_z, (N_RING - 1) - my_z)

        def pos_coords(p):
            p = p % N_RING
            ny = jnp.where(p < 4, 0, 1)
            nz = jnp.where(p < 4, p, (N_RING - 1) - p)
            return ny, nz

        fy, fz = pos_coords(r + 1)
        by, bz = pos_coords(r - 1)
        fwd = (my_x, fy, fz)
        bwd = (my_x, by, bz)

        off_s = my_x * 176
        off_sp = (1 - my_x) * 176
        off_xs = my_x * 336

        barrier_sem = pltpu.get_barrier_semaphore()
        for nbr in (partner, fwd, bwd):
            pl.semaphore_signal(
                barrier_sem, inc=1,
                device_id=nbr, device_id_type=pl.DeviceIdType.MESH,
            )
        pl.semaphore_wait(barrier_sem, 3)

        row0 = r * MC

        p1_rdmas = []
        for s in range(2):
            p1_rdmas.append(pltpu.make_async_remote_copy(
                src_ref=x_ref.at[pl.ds(row0 + off_sp + s * SUB, SUB)],
                dst_ref=p1_buf.at[pl.ds(s * SUB, SUB)],
                send_sem=p1_send.at[s], recv_sem=p1_recv.at[s],
                device_id=partner, device_id_type=pl.DeviceIdType.MESH,
            ))
            p1_rdmas[s].start()

        def mk_hop(h, s, sign, sems_s, sems_r, dev):
            c = (r + sign * (h - 1)) % N_RING
            rows = pl.ds(c * MC + off_s + s * SUB, SUB)
            i = 2 * (h - 1) + s
            return pltpu.make_async_remote_copy(
                src_ref=out_ref.at[rows], dst_ref=out_ref.at[rows],
                send_sem=sems_s.at[i], recv_sem=sems_r.at[i],
                device_id=dev, device_id_type=pl.DeviceIdType.MESH,
            )

        def mk_xfwd(slot, c):
            rows = pl.ds(c * MC + off_xs, PX)
            return pltpu.make_async_remote_copy(
                src_ref=out_ref.at[rows], dst_ref=out_ref.at[rows],
                send_sem=xf_send.at[slot], recv_sem=xf_recv.at[slot],
                device_id=partner, device_id_type=pl.DeviceIdType.MESH,
            )

        fwd_rdmas = {}
        bwd_rdmas = {}
        x_rdmas = []

        for s in range(2):
            p1_rdmas[s].wait_recv()
            out_ref[pl.ds(row0 + off_s + s * SUB, SUB), :] = (
                x_ref[pl.ds(row0 + off_s + s * SUB, SUB), :]
                + p1_buf[pl.ds(s * SUB, SUB), :]
            )
            fwd_rdmas[(1, s)] = mk_hop(1, s, -1, fwd_send, fwd_recv, fwd)
            fwd_rdmas[(1, s)].start()
            bwd_rdmas[(1, s)] = mk_hop(1, s, +1, bwd_send, bwd_recv, bwd)
            bwd_rdmas[(1, s)].start()
        x_rdmas.append(mk_xfwd(0, r))
        x_rdmas[-1].start()

        for rd in p1_rdmas:
            rd.wait_send()

        for h in range(1, NH + 1):
            subs_f = (0, 1) if h < NH else (0,)
            subs_b = (0, 1) if h < NH else (1,)
            for s in subs_f:
                fwd_rdmas[(h, s)].wait_recv()
                if h < NH and (h + 1 < NH or s == 0):
                    fwd_rdmas[(h + 1, s)] = mk_hop(
                        h + 1, s, -1, fwd_send, fwd_recv, fwd)
                    fwd_rdmas[(h + 1, s)].start()
            if h < NH:
                x_rdmas.append(mk_xfwd(2 * h - 1, (r - h) % N_RING))
                x_rdmas[-1].start()
            for s in subs_b:
                bwd_rdmas[(h, s)].wait_recv()
                if h < NH and (h + 1 < NH or s == 1):
                    bwd_rdmas[(h + 1, s)] = mk_hop(
                        h + 1, s, +1, bwd_send, bwd_recv, bwd)
                    bwd_rdmas[(h + 1, s)].start()
            if h < NH:
                x_rdmas.append(mk_xfwd(2 * h, (r + h) % N_RING))
                x_rdmas[-1].start()
            for s in subs_f:
                fwd_rdmas[(h, s)].wait_send()
            for s in subs_b:
                bwd_rdmas[(h, s)].wait_send()

        x_rdmas.append(mk_xfwd(7, (r + NH) % N_RING))
        x_rdmas[-1].start()

        for rd in x_rdmas:
            rd.wait_recv()
            rd.wait_send()

    return pl.pallas_call(
        body,
        out_shape=jax.ShapeDtypeStruct((m, n), x.dtype),
        in_specs=[pl.BlockSpec(memory_space=pltpu.VMEM)],
        out_specs=pl.BlockSpec(memory_space=pltpu.VMEM),
        scratch_shapes=[
            pltpu.VMEM((PS, n), x.dtype),
            pltpu.SemaphoreType.DMA((2,)),
            pltpu.SemaphoreType.DMA((2,)),
            pltpu.SemaphoreType.DMA((2 * NH,)),
            pltpu.SemaphoreType.DMA((2 * NH,)),
            pltpu.SemaphoreType.DMA((2 * NH,)),
            pltpu.SemaphoreType.DMA((2 * NH,)),
            pltpu.SemaphoreType.DMA((N_RING,)),
            pltpu.SemaphoreType.DMA((N_RING,)),
        ],
        compiler_params=pltpu.CompilerParams(
            collective_id=0,
            vmem_limit_bytes=100 * 1024 * 1024,
        ),
    )(x)


# device time: 108161 ns/iter; 1.0121x vs baseline; 1.0101x over previous
import jax
import jax.numpy as jnp
from jax import lax
from jax.experimental import pallas as pl
from jax.experimental.pallas import tpu as pltpu

N_RING = 8
MC = 512
PF = 192
NSUB = 3
SB = PF // NSUB
PX = 128
NHOP = N_RING - 1


def kernel(x):
    m, n = x.shape
    assert m == N_RING * MC

    def body(x_ref, out_ref, p1_buf,
             p1_send, p1_recv, fwd_send, fwd_recv,
             bwd_send, bwd_recv, xf_send, xf_recv):
        my_x = lax.axis_index("x")
        my_y = lax.axis_index("y")
        my_z = lax.axis_index("z")
        partner = (1 - my_x, my_y, my_z)

        r = jnp.where(my_y == 0, my_z, (N_RING - 1) - my_z)

        def pos_coords(p):
            p = p % N_RING
            ny = jnp.where(p < 4, 0, 1)
            nz = jnp.where(p < 4, p, (N_RING - 1) - p)
            return ny, nz

        fy, fz = pos_coords(r + 1)
        by, bz = pos_coords(r - 1)
        fwd = (my_x, fy, fz)
        bwd = (my_x, by, bz)

        off_f = my_x * 320
        off_b = 192 - my_x * 64
        off_fp = (1 - my_x) * 320
        off_bp = 192 - (1 - my_x) * 64
        off_xs = my_x * 384

        barrier_sem = pltpu.get_barrier_semaphore()
        for nbr in (partner, fwd, bwd):
            pl.semaphore_signal(
                barrier_sem, inc=1,
                device_id=nbr, device_id_type=pl.DeviceIdType.MESH,
            )
        pl.semaphore_wait(barrier_sem, 3)

        row0 = r * MC

        p1_rdmas = []
        for s in range(NSUB):
            p1_rdmas.append(pltpu.make_async_remote_copy(
                src_ref=x_ref.at[pl.ds(row0 + off_fp + s * SB, SB)],
                dst_ref=p1_buf.at[pl.ds(s * SB, SB)],
                send_sem=p1_send.at[s], recv_sem=p1_recv.at[s],
                device_id=partner, device_id_type=pl.DeviceIdType.MESH,
            ))
        p1_rdmas.append(pltpu.make_async_remote_copy(
            src_ref=x_ref.at[pl.ds(row0 + off_bp, PF)],
            dst_ref=p1_buf.at[pl.ds(PF, PF)],
            send_sem=p1_send.at[NSUB], recv_sem=p1_recv.at[NSUB],
            device_id=partner, device_id_type=pl.DeviceIdType.MESH,
        ))
        for rd in p1_rdmas:
            rd.start()

        def mk_hop(h, s, off, sems_s, sems_r, dev):
            if off is off_f:
                c = (r - (h - 1)) % N_RING
            else:
                c = (r + (h - 1)) % N_RING
            rows = pl.ds(c * MC + off + s * SB, SB)
            i = NSUB * (h - 1) + s
            return pltpu.make_async_remote_copy(
                src_ref=out_ref.at[rows], dst_ref=out_ref.at[rows],
                send_sem=sems_s.at[i], recv_sem=sems_r.at[i],
                device_id=dev, device_id_type=pl.DeviceIdType.MESH,
            )

        def mk_xfwd(h):
            c = (r - h) % N_RING
            rows = pl.ds(c * MC + off_xs, PX)
            return pltpu.make_async_remote_copy(
                src_ref=out_ref.at[rows], dst_ref=out_ref.at[rows],
                send_sem=xf_send.at[h], recv_sem=xf_recv.at[h],
                device_id=partner, device_id_type=pl.DeviceIdType.MESH,
            )

        fwd_rdmas = {}
        bwd_rdmas = {}
        x_rdmas = {}

        for s in range(NSUB):
            p1_rdmas[s].wait_recv()
            out_ref[pl.ds(row0 + off_f + s * SB, SB), :] = (
                x_ref[pl.ds(row0 + off_f + s * SB, SB), :]
                + p1_buf[pl.ds(s * SB, SB), :]
            )
            fwd_rdmas[(1, s)] = mk_hop(1, s, off_f, fwd_send, fwd_recv, fwd)
            fwd_rdmas[(1, s)].start()
        x_rdmas[0] = mk_xfwd(0)
        x_rdmas[0].start()

        p1_rdmas[NSUB].wait_recv()
        out_ref[pl.ds(row0 + off_b, PF), :] = (
            x_ref[pl.ds(row0 + off_b, PF), :] + p1_buf[pl.ds(PF, PF), :]
        )
        for s in range(NSUB):
            bwd_rdmas[(1, s)] = mk_hop(1, s, off_b, bwd_send, bwd_recv, bwd)
            bwd_rdmas[(1, s)].start()

        for rd in p1_rdmas:
            rd.wait_send()

        for h in range(1, N_RING):
            for s in range(NSUB):
                fwd_rdmas[(h, s)].wait_recv()
                if h < NHOP:
                    fwd_rdmas[(h + 1, s)] = mk_hop(
                        h + 1, s, off_f, fwd_send, fwd_recv, fwd)
                    fwd_rdmas[(h + 1, s)].start()
            x_rdmas[h] = mk_xfwd(h)
            x_rdmas[h].start()
            for s in range(NSUB):
                bwd_rdmas[(h, s)].wait_recv()
                if h < NHOP:
                    bwd_rdmas[(h + 1, s)] = mk_hop(
                        h + 1, s, off_b, bwd_send, bwd_recv, bwd)
                    bwd_rdmas[(h + 1, s)].start()
            for s in range(NSUB):
                fwd_rdmas[(h, s)].wait_send()
                bwd_rdmas[(h, s)].wait_send()

        for h in range(N_RING):
            x_rdmas[h].wait_recv()
            x_rdmas[h].wait_send()

    return pl.pallas_call(
        body,
        out_shape=jax.ShapeDtypeStruct((m, n), x.dtype),
        in_specs=[pl.BlockSpec(memory_space=pltpu.VMEM)],
        out_specs=pl.BlockSpec(memory_space=pltpu.VMEM),
        scratch_shapes=[
            pltpu.VMEM((2 * PF, n), x.dtype),
            pltpu.SemaphoreType.DMA((NSUB + 1,)),
            pltpu.SemaphoreType.DMA((NSUB + 1,)),
            pltpu.SemaphoreType.DMA((NSUB * NHOP,)),
            pltpu.SemaphoreType.DMA((NSUB * NHOP,)),
            pltpu.SemaphoreType.DMA((NSUB * NHOP,)),
            pltpu.SemaphoreType.DMA((NSUB * NHOP,)),
            pltpu.SemaphoreType.DMA((N_RING,)),
            pltpu.SemaphoreType.DMA((N_RING,)),
        ],
        compiler_params=pltpu.CompilerParams(
            collective_id=0,
            vmem_limit_bytes=100 * 1024 * 1024,
        ),
    )(x)
